# baseline (speedup 1.0000x reference)
import functools

import jax
import jax.numpy as jnp
from jax import lax
from jax.experimental import pallas as pl
from jax.experimental.pallas import tpu as pltpu
from jax.experimental.pallas import tpu_sc as plsc

N = 10000
H = 128
E = 320000
NUM_CORES = 1
NUM_SUBCORES = 16
NUM_TILES = NUM_CORES * NUM_SUBCORES
BLK = 128
NBLKS = E // BLK
NB_BASE = NBLKS // NUM_TILES
NB_EXTRA = NBLKS - NB_BASE * NUM_TILES
NB_MAX = NB_BASE + 1
E_TILE = NB_MAX * BLK
E_BASE = NB_BASE * BLK

N_PAD = 10240
COL_BLK = N_PAD // 8
INIT_CHUNK = N_PAD // NUM_SUBCORES


def _node_scores_tc(h, w_edge, w_node, b_edge, b_node):
    def body(h_ref, we_ref, wn_ref, be_ref, bn_ref, o_ref):
        w3 = jnp.concatenate(
            [we_ref[...].reshape(2, H), wn_ref[...].reshape(1, H)], axis=0)
        s = lax.dot_general(
            w3, h_ref[...], (((1,), (1,)), ((), ())),
            preferred_element_type=jnp.float32)
        row = lax.broadcasted_iota(jnp.int32, (3, 1), 0)
        b3 = jnp.where(row == 1, be_ref[0, 0], 0.0) + jnp.where(
            row == 2, bn_ref[0, 0], 0.0)
        o_ref[0:3, :] = s + b3
        o_ref[3:4, :] = jnp.zeros((1, N), jnp.float32)

    return pl.pallas_call(
        body,
        in_specs=[
            pl.BlockSpec((N, H), lambda: (0, 0)),
            pl.BlockSpec((2 * H,), lambda: (0,)),
            pl.BlockSpec((H,), lambda: (0,)),
            pl.BlockSpec(memory_space=pltpu.SMEM),
            pl.BlockSpec(memory_space=pltpu.SMEM),
        ],
        out_specs=pl.BlockSpec((4, N), lambda: (0, 0)),
        out_shape=jax.ShapeDtypeStruct((4, N), jnp.float32),
    )(h, w_edge, w_node,
      b_edge.reshape(1, 1).astype(jnp.float32),
      b_node.reshape(1, 1).astype(jnp.float32))


def _edge_accumulate_sc(s4n, ei):
    mesh = plsc.VectorSubcoreMesh(
        core_axis_name="c", subcore_axis_name="s", num_cores=NUM_CORES)

    @functools.partial(
        pl.kernel,
        out_type=jax.ShapeDtypeStruct((NUM_CORES, N), jnp.float32),
        mesh=mesh,
        compiler_params=pltpu.CompilerParams(needs_layout_passes=False),
        scratch_types=[
            pltpu.VMEM((2, E_TILE), jnp.int32),
            pltpu.VMEM((E_TILE,), jnp.int32),
            pltpu.VMEM((E_TILE,), jnp.float32),
            pltpu.VMEM((N,), jnp.float32),
            pltpu.VMEM((N,), jnp.float32),
            pltpu.VMEM_SHARED((N,), jnp.float32),
        ],
    )
    def k(s_hbm, ei_hbm, out_hbm,
          ei_v, dst_v, vals_v, s1_v, s2_v, acc_sh):
        c = lax.axis_index("c")
        s = lax.axis_index("s")
        wid = c * NUM_SUBCORES + s
        has_extra = wid < NB_EXTRA
        base = (wid * NB_BASE + jnp.minimum(wid, NB_EXTRA)) * BLK

        pltpu.sync_copy(ei_hbm.at[:, pl.ds(base, E_BASE)],
                        ei_v.at[:, pl.ds(0, E_BASE)])

        @pl.when(has_extra)
        def _():
            pltpu.sync_copy(ei_hbm.at[:, pl.ds(base + E_BASE, BLK)],
                            ei_v.at[:, pl.ds(E_BASE, BLK)])

        @pl.when(jnp.logical_not(has_extra))
        def _():
            for u in range(BLK // 16):
                idx = u * 16 + lax.iota(jnp.int32, 16)
                ei_v[0, pl.ds(E_BASE + u * 16, 16)] = idx
                ei_v[1, pl.ds(E_BASE + u * 16, 16)] = idx

        pltpu.sync_copy(s_hbm.at[0], s1_v)
        pltpu.sync_copy(s_hbm.at[1], s2_v)

        @pl.when(s == 0)
        def _():
            pltpu.sync_copy(s_hbm.at[3], acc_sh)

        plsc.subcore_barrier()

        def chunk(i, carry):
            b0 = i * BLK
            for u in range(BLK // 16):
                o = b0 + u * 16
                si = ei_v[0, pl.ds(o, 16)]
                di = ei_v[1, pl.ds(o, 16)]
                g = (plsc.load_gather(s1_v, [si])
                     + plsc.load_gather(s2_v, [di]))
                vals_v[pl.ds(o, 16)] = g
                dst_v[pl.ds(o, 16)] = di
            return carry

        lax.fori_loop(0, NB_MAX, chunk, 0)

        @pl.when(jnp.logical_not(has_extra))
        def _():
            zero = jnp.zeros((16,), jnp.float32)
            for u in range(BLK // 16):
                vals_v[pl.ds(E_BASE + u * 16, 16)] = zero

        pltpu.sync_copy(vals_v, acc_sh.at[dst_v], add=True)
        plsc.subcore_barrier()

        @pl.when(s == 0)
        def _():
            pltpu.sync_copy(acc_sh, out_hbm.at[c])

    return k(s4n, ei)


def _combine_softmax_tc(parts, s4n):
    def body(p_ref, s_ref, o_ref):
        combined = p_ref[0:1, 0:N] + s_ref[2:3, 0:N]
        if NUM_CORES > 1:
            combined = combined + p_ref[1:2, 0:N]
        m = jnp.max(combined)
        e = jnp.exp(combined - m)
        o_ref[...] = e / jnp.sum(e)

    return pl.pallas_call(
        body,
        out_shape=jax.ShapeDtypeStruct((1, N), jnp.float32),
    )(parts, s4n)


def kernel(h, edge_index, W_edge, b_edge, W_node, b_node):
    h = h.astype(jnp.float32)
    ei = edge_index.astype(jnp.int32)

    s4n = _node_scores_tc(h, W_edge, W_node, b_edge, b_node)
    parts = _edge_accumulate_sc(s4n, ei)
    out = _combine_softmax_tc(parts, s4n)
    return out.reshape(N)


# speedup vs baseline: 1.2294x; 1.2294x over previous
import functools

import jax
import jax.numpy as jnp
from jax import lax
from jax.experimental import pallas as pl
from jax.experimental.pallas import tpu as pltpu
from jax.experimental.pallas import tpu_sc as plsc

N = 10000
H = 128
E = 320000
NUM_CORES = 2
NUM_SUBCORES = 16
NUM_TILES = NUM_CORES * NUM_SUBCORES
BLK = 128
NBLKS = E // BLK
NB_BASE = NBLKS // NUM_TILES
NB_EXTRA = NBLKS - NB_BASE * NUM_TILES
NB_MAX = NB_BASE + 1
E_TILE = NB_MAX * BLK
E_BASE = NB_BASE * BLK

N_PAD = 10240
COL_BLK = N_PAD // 8
INIT_CHUNK = N_PAD // NUM_SUBCORES


def _node_scores_tc(h, w_edge, w_node, b_edge, b_node):
    def body(h_ref, we_ref, wn_ref, be_ref, bn_ref, o_ref):
        w3 = jnp.concatenate(
            [we_ref[...].reshape(2, H), wn_ref[...].reshape(1, H)], axis=0)
        s = lax.dot_general(
            w3, h_ref[...], (((1,), (1,)), ((), ())),
            preferred_element_type=jnp.float32)
        row = lax.broadcasted_iota(jnp.int32, (3, 1), 0)
        b3 = jnp.where(row == 1, be_ref[0, 0], 0.0) + jnp.where(
            row == 2, bn_ref[0, 0], 0.0)
        o_ref[0:3, :] = s + b3
        o_ref[3:4, :] = jnp.zeros((1, N), jnp.float32)

    return pl.pallas_call(
        body,
        in_specs=[
            pl.BlockSpec((N, H), lambda: (0, 0)),
            pl.BlockSpec((2 * H,), lambda: (0,)),
            pl.BlockSpec((H,), lambda: (0,)),
            pl.BlockSpec(memory_space=pltpu.SMEM),
            pl.BlockSpec(memory_space=pltpu.SMEM),
        ],
        out_specs=pl.BlockSpec((4, N), lambda: (0, 0)),
        out_shape=jax.ShapeDtypeStruct((4, N), jnp.float32),
    )(h, w_edge, w_node,
      b_edge.reshape(1, 1).astype(jnp.float32),
      b_node.reshape(1, 1).astype(jnp.float32))


def _edge_accumulate_sc(s4n, ei):
    mesh = plsc.VectorSubcoreMesh(
        core_axis_name="c", subcore_axis_name="s", num_cores=NUM_CORES)

    @functools.partial(
        pl.kernel,
        out_type=jax.ShapeDtypeStruct((NUM_CORES, N), jnp.float32),
        mesh=mesh,
        compiler_params=pltpu.CompilerParams(needs_layout_passes=False),
        scratch_types=[
            pltpu.VMEM((2, E_TILE), jnp.int32),
            pltpu.VMEM((E_TILE,), jnp.int32),
            pltpu.VMEM((E_TILE,), jnp.float32),
            pltpu.VMEM((N,), jnp.float32),
            pltpu.VMEM((N,), jnp.float32),
            pltpu.VMEM_SHARED((N,), jnp.float32),
            pltpu.SemaphoreType.DMA,
        ],
    )
    def k(s_hbm, ei_hbm, out_hbm,
          ei_v, dst_v, vals_v, s1_v, s2_v, acc_sh, sem):
        c = lax.axis_index("c")
        s = lax.axis_index("s")
        wid = c * NUM_SUBCORES + s
        has_extra = wid < NB_EXTRA
        base = (wid * NB_BASE + jnp.minimum(wid, NB_EXTRA)) * BLK

        cp_ei = pltpu.async_copy(ei_hbm.at[:, pl.ds(base, E_BASE)],
                                 ei_v.at[:, pl.ds(0, E_BASE)], sem)
        cp_s1 = pltpu.async_copy(s_hbm.at[0], s1_v, sem)
        cp_s2 = pltpu.async_copy(s_hbm.at[1], s2_v, sem)

        @pl.when(has_extra)
        def _():
            pltpu.sync_copy(ei_hbm.at[:, pl.ds(base + E_BASE, BLK)],
                            ei_v.at[:, pl.ds(E_BASE, BLK)])

        @pl.when(jnp.logical_not(has_extra))
        def _():
            for u in range(BLK // 16):
                idx = u * 16 + lax.iota(jnp.int32, 16)
                ei_v[0, pl.ds(E_BASE + u * 16, 16)] = idx
                ei_v[1, pl.ds(E_BASE + u * 16, 16)] = idx

        @pl.when(s == 0)
        def _():
            pltpu.sync_copy(s_hbm.at[3], acc_sh)

        cp_ei.wait()
        cp_s1.wait()
        cp_s2.wait()
        plsc.subcore_barrier()

        def chunk(i, carry):
            b0 = i * BLK
            for u in range(BLK // 16):
                o = b0 + u * 16
                si = ei_v[0, pl.ds(o, 16)]
                di = ei_v[1, pl.ds(o, 16)]
                g = (plsc.load_gather(s1_v, [si])
                     + plsc.load_gather(s2_v, [di]))
                vals_v[pl.ds(o, 16)] = g
                dst_v[pl.ds(o, 16)] = di
            return carry

        lax.fori_loop(0, NB_MAX, chunk, 0)

        @pl.when(jnp.logical_not(has_extra))
        def _():
            zero = jnp.zeros((16,), jnp.float32)
            for u in range(BLK // 16):
                vals_v[pl.ds(E_BASE + u * 16, 16)] = zero

        pltpu.sync_copy(vals_v, acc_sh.at[dst_v], add=True)
        plsc.subcore_barrier()

        @pl.when(s == 0)
        def _():
            pltpu.sync_copy(acc_sh, out_hbm.at[c])

    return k(s4n, ei)


def _combine_softmax_tc(parts, s4n):
    def body(p_ref, s_ref, o_ref):
        combined = p_ref[0:1, 0:N] + s_ref[2:3, 0:N]
        if NUM_CORES > 1:
            combined = combined + p_ref[1:2, 0:N]
        m = jnp.max(combined)
        e = jnp.exp(combined - m)
        o_ref[...] = e / jnp.sum(e)

    return pl.pallas_call(
        body,
        out_shape=jax.ShapeDtypeStruct((1, N), jnp.float32),
    )(parts, s4n)


def kernel(h, edge_index, W_edge, b_edge, W_node, b_node):
    h = h.astype(jnp.float32)
    ei = edge_index.astype(jnp.int32)

    s4n = _node_scores_tc(h, W_edge, W_node, b_edge, b_node)
    parts = _edge_accumulate_sc(s4n, ei)
    out = _combine_softmax_tc(parts, s4n)
    return out.reshape(N)


# speedup vs baseline: 1.2707x; 1.0336x over previous
import functools

import jax
import jax.numpy as jnp
from jax import lax
from jax.experimental import pallas as pl
from jax.experimental.pallas import tpu as pltpu
from jax.experimental.pallas import tpu_sc as plsc

N = 10000
H = 128
E = 320000
NUM_CORES = 2
NUM_SUBCORES = 16
NUM_TILES = NUM_CORES * NUM_SUBCORES
BLK = 128
NBLKS = E // BLK
NB_BASE = NBLKS // NUM_TILES
NB_EXTRA = NBLKS - NB_BASE * NUM_TILES
NB_MAX = NB_BASE + 1
E_TILE = NB_MAX * BLK
E_BASE = NB_BASE * BLK
NB_A = 40

N_PAD = 10240
COL_BLK = N_PAD // 8
INIT_CHUNK = N_PAD // NUM_SUBCORES


def _node_scores_tc(h, w_edge, w_node, b_edge, b_node):
    def body(h_ref, we_ref, wn_ref, be_ref, bn_ref, o_ref):
        w3 = jnp.concatenate(
            [we_ref[...].reshape(2, H), wn_ref[...].reshape(1, H)], axis=0)
        s = lax.dot_general(
            w3, h_ref[...], (((1,), (1,)), ((), ())),
            preferred_element_type=jnp.float32)
        row = lax.broadcasted_iota(jnp.int32, (3, 1), 0)
        b3 = jnp.where(row == 1, be_ref[0, 0], 0.0) + jnp.where(
            row == 2, bn_ref[0, 0], 0.0)
        o_ref[0:3, :] = s + b3
        o_ref[3:4, :] = jnp.zeros((1, N), jnp.float32)

    return pl.pallas_call(
        body,
        in_specs=[
            pl.BlockSpec((N, H), lambda: (0, 0)),
            pl.BlockSpec((2 * H,), lambda: (0,)),
            pl.BlockSpec((H,), lambda: (0,)),
            pl.BlockSpec(memory_space=pltpu.SMEM),
            pl.BlockSpec(memory_space=pltpu.SMEM),
        ],
        out_specs=pl.BlockSpec((4, N), lambda: (0, 0)),
        out_shape=jax.ShapeDtypeStruct((4, N), jnp.float32),
    )(h, w_edge, w_node,
      b_edge.reshape(1, 1).astype(jnp.float32),
      b_node.reshape(1, 1).astype(jnp.float32))


def _edge_accumulate_sc(s4n, ei):
    mesh = plsc.VectorSubcoreMesh(
        core_axis_name="c", subcore_axis_name="s", num_cores=NUM_CORES)

    @functools.partial(
        pl.kernel,
        out_type=jax.ShapeDtypeStruct((NUM_CORES, N), jnp.float32),
        mesh=mesh,
        compiler_params=pltpu.CompilerParams(needs_layout_passes=False),
        scratch_types=[
            pltpu.VMEM((2, E_TILE), jnp.int32),
            pltpu.VMEM((E_TILE,), jnp.int32),
            pltpu.VMEM((E_TILE,), jnp.float32),
            pltpu.VMEM((N,), jnp.float32),
            pltpu.VMEM((N,), jnp.float32),
            pltpu.VMEM_SHARED((N,), jnp.float32),
            pltpu.SemaphoreType.DMA,
            pltpu.SemaphoreType.DMA,
        ],
    )
    def k(s_hbm, ei_hbm, out_hbm,
          ei_v, dst_v, vals_v, s1_v, s2_v, acc_sh, sem, sem2):
        c = lax.axis_index("c")
        s = lax.axis_index("s")
        wid = c * NUM_SUBCORES + s
        has_extra = wid < NB_EXTRA
        base = (wid * NB_BASE + jnp.minimum(wid, NB_EXTRA)) * BLK

        cp_ei = pltpu.async_copy(ei_hbm.at[:, pl.ds(base, E_BASE)],
                                 ei_v.at[:, pl.ds(0, E_BASE)], sem)
        cp_s1 = pltpu.async_copy(s_hbm.at[0], s1_v, sem)
        cp_s2 = pltpu.async_copy(s_hbm.at[1], s2_v, sem)

        @pl.when(has_extra)
        def _():
            pltpu.sync_copy(ei_hbm.at[:, pl.ds(base + E_BASE, BLK)],
                            ei_v.at[:, pl.ds(E_BASE, BLK)])

        @pl.when(jnp.logical_not(has_extra))
        def _():
            for u in range(BLK // 16):
                idx = u * 16 + lax.iota(jnp.int32, 16)
                ei_v[0, pl.ds(E_BASE + u * 16, 16)] = idx
                ei_v[1, pl.ds(E_BASE + u * 16, 16)] = idx

        @pl.when(s == 0)
        def _():
            pltpu.sync_copy(s_hbm.at[3], acc_sh)

        cp_ei.wait()
        cp_s1.wait()
        cp_s2.wait()
        plsc.subcore_barrier()

        def chunk(i, carry):
            b0 = i * BLK
            for u in range(BLK // 16):
                o = b0 + u * 16
                si = ei_v[0, pl.ds(o, 16)]
                di = ei_v[1, pl.ds(o, 16)]
                g = (plsc.load_gather(s1_v, [si])
                     + plsc.load_gather(s2_v, [di]))
                vals_v[pl.ds(o, 16)] = g
                dst_v[pl.ds(o, 16)] = di
            return carry

        lax.fori_loop(0, NB_A, chunk, 0)
        cp_sc1 = pltpu.async_copy(
            vals_v.at[pl.ds(0, NB_A * BLK)],
            acc_sh.at[dst_v.at[pl.ds(0, NB_A * BLK)]], sem2, add=True)

        lax.fori_loop(NB_A, NB_MAX, chunk, 0)

        @pl.when(jnp.logical_not(has_extra))
        def _():
            zero = jnp.zeros((16,), jnp.float32)
            for u in range(BLK // 16):
                vals_v[pl.ds(E_BASE + u * 16, 16)] = zero

        E_REST = (NB_MAX - NB_A) * BLK
        pltpu.sync_copy(vals_v.at[pl.ds(NB_A * BLK, E_REST)],
                        acc_sh.at[dst_v.at[pl.ds(NB_A * BLK, E_REST)]],
                        add=True)
        cp_sc1.wait()
        plsc.subcore_barrier()

        @pl.when(s == 0)
        def _():
            pltpu.sync_copy(acc_sh, out_hbm.at[c])

    return k(s4n, ei)


def _combine_softmax_tc(parts, s4n):
    def body(p_ref, s_ref, o_ref):
        combined = p_ref[0:1, 0:N] + s_ref[2:3, 0:N]
        if NUM_CORES > 1:
            combined = combined + p_ref[1:2, 0:N]
        m = jnp.max(combined)
        e = jnp.exp(combined - m)
        o_ref[...] = e / jnp.sum(e)

    return pl.pallas_call(
        body,
        out_shape=jax.ShapeDtypeStruct((1, N), jnp.float32),
    )(parts, s4n)


def kernel(h, edge_index, W_edge, b_edge, W_node, b_node):
    h = h.astype(jnp.float32)
    ei = edge_index.astype(jnp.int32)

    s4n = _node_scores_tc(h, W_edge, W_node, b_edge, b_node)
    parts = _edge_accumulate_sc(s4n, ei)
    out = _combine_softmax_tc(parts, s4n)
    return out.reshape(N)


# speedup vs baseline: 1.2926x; 1.0173x over previous
import functools

import jax
import jax.numpy as jnp
from jax import lax
from jax.experimental import pallas as pl
from jax.experimental.pallas import tpu as pltpu
from jax.experimental.pallas import tpu_sc as plsc

N = 10000
H = 128
E = 320000
NUM_CORES = 2
NUM_SUBCORES = 16
NUM_TILES = NUM_CORES * NUM_SUBCORES
BLK = 128
NBLKS = E // BLK
NB_BASE = NBLKS // NUM_TILES
NB_EXTRA = NBLKS - NB_BASE * NUM_TILES
NB_MAX = NB_BASE + 1
E_TILE = NB_MAX * BLK
E_BASE = NB_BASE * BLK
NB_A = 40

N_PAD = 10240
COL_BLK = N_PAD // 8
INIT_CHUNK = N_PAD // NUM_SUBCORES


def _node_scores_tc(h, w_edge, w_node, b_edge, b_node):
    def body(h_ref, we_ref, wn_ref, be_ref, bn_ref, o_ref):
        w3 = jnp.concatenate(
            [we_ref[...].reshape(2, H), wn_ref[...].reshape(1, H)], axis=0)
        s = lax.dot_general(
            w3, h_ref[...], (((1,), (1,)), ((), ())),
            preferred_element_type=jnp.float32)
        row = lax.broadcasted_iota(jnp.int32, (3, 1), 0)
        b3 = jnp.where(row == 1, be_ref[0, 0], 0.0) + jnp.where(
            row == 2, bn_ref[0, 0], 0.0)
        o_ref[0:3, :] = s + b3
        o_ref[3:4, :] = jnp.zeros((1, N), jnp.float32)

    return pl.pallas_call(
        body,
        in_specs=[
            pl.BlockSpec((N, H), lambda: (0, 0)),
            pl.BlockSpec((2 * H,), lambda: (0,)),
            pl.BlockSpec((H,), lambda: (0,)),
            pl.BlockSpec(memory_space=pltpu.SMEM),
            pl.BlockSpec(memory_space=pltpu.SMEM),
        ],
        out_specs=pl.BlockSpec((4, N), lambda: (0, 0)),
        out_shape=jax.ShapeDtypeStruct((4, N), jnp.float32),
    )(h, w_edge, w_node,
      b_edge.reshape(1, 1).astype(jnp.float32),
      b_node.reshape(1, 1).astype(jnp.float32))


def _edge_accumulate_sc(s4n, ei):
    mesh = plsc.VectorSubcoreMesh(
        core_axis_name="c", subcore_axis_name="s", num_cores=NUM_CORES)

    @functools.partial(
        pl.kernel,
        out_type=jax.ShapeDtypeStruct((NUM_CORES, N), jnp.float32),
        mesh=mesh,
        compiler_params=pltpu.CompilerParams(needs_layout_passes=False),
        scratch_types=[
            pltpu.VMEM((2, E_TILE), jnp.int32),
            pltpu.VMEM((E_TILE,), jnp.int32),
            pltpu.VMEM((E_TILE,), jnp.float32),
            pltpu.VMEM((N,), jnp.float32),
            pltpu.VMEM((N,), jnp.float32),
            pltpu.VMEM_SHARED((N,), jnp.float32),
            pltpu.SemaphoreType.DMA,
            pltpu.SemaphoreType.DMA,
        ],
    )
    def k(s_hbm, ei_hbm, out_hbm,
          ei_v, dst_v, vals_v, s1_v, s2_v, acc_sh, sem, sem2):
        c = lax.axis_index("c")
        s = lax.axis_index("s")
        wid = c * NUM_SUBCORES + s
        has_extra = wid < NB_EXTRA
        base = (wid * NB_BASE + jnp.minimum(wid, NB_EXTRA)) * BLK

        cp_ei = pltpu.async_copy(ei_hbm.at[:, pl.ds(base, E_BASE)],
                                 ei_v.at[:, pl.ds(0, E_BASE)], sem)
        cp_s1 = pltpu.async_copy(s_hbm.at[0], s1_v, sem)
        cp_s2 = pltpu.async_copy(s_hbm.at[1], s2_v, sem)

        @pl.when(has_extra)
        def _():
            pltpu.sync_copy(ei_hbm.at[:, pl.ds(base + E_BASE, BLK)],
                            ei_v.at[:, pl.ds(E_BASE, BLK)])

        @pl.when(jnp.logical_not(has_extra))
        def _():
            for u in range(BLK // 16):
                idx = u * 16 + lax.iota(jnp.int32, 16)
                ei_v[0, pl.ds(E_BASE + u * 16, 16)] = idx
                ei_v[1, pl.ds(E_BASE + u * 16, 16)] = idx

        @pl.when(s == 0)
        def _():
            pltpu.sync_copy(s_hbm.at[3], acc_sh)

        cp_ei.wait()
        cp_s1.wait()
        cp_s2.wait()
        plsc.subcore_barrier()

        def chunk(i, carry):
            b0 = i * BLK
            for u in range(BLK // 16):
                o = b0 + u * 16
                si = ei_v[0, pl.ds(o, 16)]
                di = ei_v[1, pl.ds(o, 16)]
                g = (plsc.load_gather(s1_v, [si])
                     + plsc.load_gather(s2_v, [di]))
                vals_v[pl.ds(o, 16)] = g
                dst_v[pl.ds(o, 16)] = di
            return carry

        pending = []
        bounds = [0, 20, 40, 60, NB_MAX]
        for b_lo, b_hi in zip(bounds[:-1], bounds[1:]):
            lax.fori_loop(b_lo, b_hi, chunk, 0)
            if b_hi == NB_MAX:
                @pl.when(jnp.logical_not(has_extra))
                def _():
                    zero = jnp.zeros((16,), jnp.float32)
                    for u in range(BLK // 16):
                        vals_v[pl.ds(E_BASE + u * 16, 16)] = zero
            o, sz = b_lo * BLK, (b_hi - b_lo) * BLK
            pending.append(pltpu.async_copy(
                vals_v.at[pl.ds(o, sz)],
                acc_sh.at[dst_v.at[pl.ds(o, sz)]], sem2, add=True))
        for cp in pending:
            cp.wait()
        plsc.subcore_barrier()

        @pl.when(s == 0)
        def _():
            pltpu.sync_copy(acc_sh, out_hbm.at[c])

    return k(s4n, ei)


def _combine_softmax_tc(parts, s4n):
    def body(p_ref, s_ref, o_ref):
        combined = p_ref[0:1, 0:N] + s_ref[2:3, 0:N]
        if NUM_CORES > 1:
            combined = combined + p_ref[1:2, 0:N]
        m = jnp.max(combined)
        e = jnp.exp(combined - m)
        o_ref[...] = e / jnp.sum(e)

    return pl.pallas_call(
        body,
        out_shape=jax.ShapeDtypeStruct((1, N), jnp.float32),
    )(parts, s4n)


def kernel(h, edge_index, W_edge, b_edge, W_node, b_node):
    h = h.astype(jnp.float32)
    ei = edge_index.astype(jnp.int32)

    s4n = _node_scores_tc(h, W_edge, W_node, b_edge, b_node)
    parts = _edge_accumulate_sc(s4n, ei)
    out = _combine_softmax_tc(parts, s4n)
    return out.reshape(N)
